# Initial kernel scaffold; baseline (speedup 1.0000x reference)
import jax
import jax.numpy as jnp
from jax.experimental import pallas as pl


def kernel(pix_to_face, bary_coords, verts_features, faces):
    raise NotImplementedError("write your pallas kernel here")


# speedup vs baseline: 25.2614x; 25.2614x over previous
import functools

import jax
import jax.numpy as jnp
from jax import lax
from jax.experimental import pallas as pl
from jax.experimental.pallas import tpu as pltpu
from jax.experimental.pallas import tpu_sc as plsc

NC, NS, L = 2, 16, 16
NW = NC * NS
IC = 128


def _shade_fn(n_pix, n_c, blk, p2f_hbm, bary_hbm, vf_hbm, faces_hbm, out_hbm,
              p2f_v, idx_v, bary_v, fverts_v, vidx_v, rows_v, out_v, sem):
    n_per_w = n_pix // NW
    n_blocks = n_per_w // blk
    wid = lax.axis_index("s") * NC + lax.axis_index("c")
    base0 = wid * n_per_w

    def do_block(b, _):
        base = base0 + b * blk
        pltpu.sync_copy(p2f_hbm.at[pl.ds(base, blk)], p2f_v)
        pltpu.sync_copy(bary_hbm.at[pl.ds(base, blk)], bary_v)

        def clamp_body(i, _):
            v = p2f_v[pl.ds(i * L, L)]
            idx_v[i // (IC // L), pl.ds((i % (IC // L)) * L, L)] = (
                jnp.maximum(v, 0))
            return _

        lax.fori_loop(0, blk // L, clamp_body, None)

        cps = [
            pltpu.async_copy(faces_hbm.at[idx_v.at[j]],
                             fverts_v.at[pl.ds(j * IC, IC)], sem)
            for j in range(blk // IC)
        ]
        for cp in cps:
            cp.wait()

        def flat_body(q, _):
            t = 16 * q + lax.iota(jnp.int32, L)
            val = plsc.load_gather(fverts_v, [t // 3, t % 3])
            vidx_v[q // (IC // L), pl.ds((q % (IC // L)) * L, L)] = val
            return _

        lax.fori_loop(0, 3 * blk // L, flat_body, None)

        cps = [
            pltpu.async_copy(vf_hbm.at[vidx_v.at[m]],
                             rows_v.at[pl.ds(m * IC, IC)], sem)
            for m in range(3 * blk // IC)
        ]
        for cp in cps:
            cp.wait()

        def px_body(i, _):
            l = i * L + lax.iota(jnp.int32, L)
            pv = p2f_v[pl.ds(i * L, L)]
            mf = jnp.where(pv >= 0, jnp.float32(1.0), jnp.float32(0.0))
            w = [
                plsc.load_gather(bary_v, [l, jnp.full((L,), j, jnp.int32)])
                * mf
                for j in range(3)
            ]
            r = [3 * l + j for j in range(3)]
            for c in range(n_c):
                cc = jnp.full((L,), c, jnp.int32)
                acc = w[0] * plsc.load_gather(rows_v, [r[0], cc])
                acc += w[1] * plsc.load_gather(rows_v, [r[1], cc])
                acc += w[2] * plsc.load_gather(rows_v, [r[2], cc])
                plsc.store_scatter(out_v, [l, cc], acc)
            return _

        lax.fori_loop(0, blk // L, px_body, None)
        pltpu.sync_copy(out_v, out_hbm.at[pl.ds(base, blk)])
        return _

    lax.fori_loop(0, n_blocks, do_block, None)


def _shade(p2f, bary, vf, faces):
    n_pix = p2f.shape[0]
    n_c = vf.shape[1]
    blk = 1024
    mesh = plsc.VectorSubcoreMesh(core_axis_name="c", subcore_axis_name="s",
                                  num_cores=NC, num_subcores=NS)
    return pl.kernel(
        functools.partial(_shade_fn, n_pix, n_c, blk),
        out_type=jax.ShapeDtypeStruct((n_pix, n_c), jnp.float32),
        mesh=mesh,
        compiler_params=pltpu.CompilerParams(needs_layout_passes=False, use_tc_tiling_on_sc=False),
        scratch_types=[
            pltpu.VMEM((blk,), jnp.int32),
            pltpu.VMEM((blk // IC, IC), jnp.int32),
            pltpu.VMEM((blk, 3), jnp.float32),
            pltpu.VMEM((blk, 16), jnp.int32),
            pltpu.VMEM((3 * blk // IC, IC), jnp.int32),
            pltpu.VMEM((3 * blk, n_c), jnp.float32),
            pltpu.VMEM((blk, n_c), jnp.float32),
            pltpu.SemaphoreType.DMA,
        ],
    )(p2f, bary, vf, faces)


def kernel(pix_to_face, bary_coords, verts_features, faces):
    b, h, w, k = pix_to_face.shape
    v, c = verts_features.shape
    n = b * h * w
    p2f = pix_to_face[..., 0].reshape(n).astype(jnp.int32)
    bary = bary_coords[:, :, :, 0, :].reshape(n, 3)
    faces32 = jnp.concatenate(
        [faces.astype(jnp.int32),
         jnp.zeros((faces.shape[0], 13), jnp.int32)], axis=1)
    out = _shade(p2f, bary, verts_features, faces32)
    return out.reshape(b, h, w, c)
